# baseline (speedup 1.0000x reference)
import functools

import jax
import jax.numpy as jnp
from jax import lax
from jax.experimental import pallas as pl
from jax.experimental.pallas import tpu as pltpu
from jax.experimental.pallas import tpu_sc as plsc

N = 10000
E = 160000
F_IN = 256
H = 128
C = 10

NC, NS = 2, 16
NW = NC * NS
K = 128
EPT = ((E + NW * K - 1) // (NW * K)) * K
E_PAD = EPT * NW
NCHUNK = EPT // K
NP = 10112
RPS = NP // NS

BROWS = 1264
GRID = NP // BROWS

_mesh = plsc.VectorSubcoreMesh(core_axis_name="c", subcore_axis_name="s")


NBUF = 2
TOTCH = E_PAD // K
CH_A = 64
CH_B = 16
CH_MAX = max(CH_A, CH_B)
E_TOT = (TOTCH + CH_MAX) * K


@functools.partial(
    pl.kernel,
    out_type=jax.ShapeDtypeStruct((NC, NP, H), jnp.float32),
    mesh=_mesh,
    scratch_types=[
        pltpu.VMEM((NCHUNK, K), jnp.int32),
        pltpu.VMEM((K, H), jnp.float32),
        pltpu.VMEM_SHARED((NP, H), jnp.float32),
    ],
)
def _sc_hist(dst_hbm, ones_hbm, zeros_hbm, out_hbm, dall, ones_v, acc):
    c = lax.axis_index("c")
    s = lax.axis_index("s")
    wid = c * NS + s
    pltpu.sync_copy(zeros_hbm.at[pl.ds(s * RPS, RPS)],
                    acc.at[pl.ds(s * RPS, RPS)])
    pltpu.sync_copy(dst_hbm.at[pl.ds(wid * NCHUNK, NCHUNK)], dall)
    pltpu.sync_copy(ones_hbm, ones_v)
    plsc.subcore_barrier()

    def body(j, carry):
        pltpu.sync_copy(ones_v, acc.at[dall.at[j]], add=True)
        return carry

    lax.fori_loop(0, NCHUNK, body, 0)
    plsc.subcore_barrier()
    pltpu.sync_copy(acc.at[pl.ds(s * RPS, RPS)],
                    out_hbm.at[c, pl.ds(s * RPS, RPS)])


@functools.partial(
    pl.kernel,
    out_type=jax.ShapeDtypeStruct((NC, NP, H), jnp.float32),
    mesh=_mesh,
    scratch_types=[
        pltpu.VMEM((CH_MAX, K), jnp.int32),
        pltpu.VMEM((CH_MAX, K), jnp.int32),
        [pltpu.VMEM((K, H), jnp.float32)] * NBUF,
        pltpu.VMEM_SHARED((NP, H), jnp.float32),
        [pltpu.SemaphoreType.DMA] * NBUF,
    ],
)
def _sc_prop(y_hbm, src_hbm, dst_hbm, zeros_hbm, out_hbm,
             sall, dall, rows, acc, sems):
    c = lax.axis_index("c")
    s = lax.axis_index("s")
    cnt = jnp.where(c == 0, CH_A, CH_B)
    base = jnp.where(c == 0, s * CH_A, NS * CH_A + s * CH_B)
    pltpu.sync_copy(src_hbm.at[pl.ds(base, CH_MAX)], sall)
    pltpu.sync_copy(dst_hbm.at[pl.ds(base, CH_MAX)], dall)
    pltpu.sync_copy(zeros_hbm.at[pl.ds(s * RPS, RPS)],
                    acc.at[pl.ds(s * RPS, RPS)])
    plsc.subcore_barrier()

    for b in range(NBUF):
        pltpu.async_copy(y_hbm.at[sall.at[b]], rows[b], sems[b])

    ngrp = cnt // NBUF

    def grp(g, carry):
        for b in range(NBUF):
            j = g * NBUF + b
            pltpu.make_async_copy(y_hbm.at[sall.at[b]], rows[b],
                                  sems[b]).wait()
            pltpu.sync_copy(rows[b], acc.at[dall.at[j]], add=True)

            @pl.when(g + 1 < ngrp)
            def _():
                pltpu.async_copy(y_hbm.at[sall.at[j + NBUF]], rows[b],
                                 sems[b])
        return carry

    lax.fori_loop(0, ngrp, grp, 0)
    plsc.subcore_barrier()
    pltpu.sync_copy(acc.at[pl.ds(s * RPS, RPS)],
                    out_hbm.at[c, pl.ds(s * RPS, RPS)])


NXU = 6


def _xmm_body(x_ref, wx_ref, xu_ref):
    x = x_ref[...]
    for k in range(NXU):
        xu_ref[k] = jnp.dot(x, wx_ref[k], preferred_element_type=jnp.float32)


def _scale1_body(xu_ref, dg0_ref, dg1_ref, y_ref, dinv_ref):
    i = pl.program_id(0)
    d = dg0_ref[0][:, 0:1] + dg1_ref[0][:, 0:1]
    rid = lax.broadcasted_iota(jnp.int32, (BROWS, 1), 0) + i * BROWS
    dv = jnp.where(rid < N, lax.rsqrt(1.0 + d), 0.0)
    y_ref[...] = dv * xu_ref[0]
    dinv_ref[...] = jnp.broadcast_to(dv, (BROWS, H))


def _layer_body(xu_ref, wh_ref, b_ref, p0_ref, p1_ref, yprev_ref, dinv_ref,
                yout_ref):
    dv = dinv_ref[...]
    h = jnp.maximum(dv * (p0_ref[0] + p1_ref[0] + yprev_ref[...]) + b_ref[...],
                    0.0)
    z = xu_ref[0] + jnp.dot(h, wh_ref[...],
                            preferred_element_type=jnp.float32)
    yout_ref[...] = dv * z


def _final_body(xu1_ref, xu2_ref, b4_ref, wl1h_ref, bl1_ref, wl2h_ref,
                bl2_ref, p0_ref, p1_ref, y4_ref, dinv_ref, out_ref):
    dv = dinv_ref[...]
    h4 = jnp.maximum(dv * (p0_ref[0] + p1_ref[0] + y4_ref[...]) + b4_ref[...],
                     0.0)
    h = jnp.maximum(
        xu1_ref[0]
        + jnp.dot(h4, wl1h_ref[...], preferred_element_type=jnp.float32)
        + bl1_ref[...], 0.0)
    o = (xu2_ref[0]
         + jnp.dot(h, wl2h_ref[...], preferred_element_type=jnp.float32)
         + bl2_ref[...])
    col = lax.broadcasted_iota(jnp.int32, (BROWS, H), 1)
    om = jnp.where(col < C, o, -jnp.inf)
    m = jnp.max(om, axis=1, keepdims=True)
    lse = jnp.log(jnp.sum(jnp.exp(om - m), axis=1, keepdims=True))
    out_ref[...] = (om - m - lse)[:, :C]


def _row_spec(cols):
    return pl.BlockSpec((BROWS, cols), lambda i: (i, 0))


def _part_spec(cols, part):
    return pl.BlockSpec((1, BROWS, cols), lambda i, p=part: (p, i, 0))


def _full_spec(shape):
    return pl.BlockSpec(shape, lambda i: tuple(0 for _ in shape))


def _xu_spec(part):
    return pl.BlockSpec((1, BROWS, H), lambda i, p=part: (p, i, 0))


def _xmm(x_p, wx):
    return pl.pallas_call(
        _xmm_body,
        grid=(GRID,),
        in_specs=[_row_spec(F_IN), _full_spec((NXU, F_IN, H))],
        out_specs=pl.BlockSpec((NXU, BROWS, H), lambda i: (0, i, 0)),
        out_shape=jax.ShapeDtypeStruct((NXU, NP, H), jnp.float32),
    )(x_p, wx)


def _scale1(xu, deg):
    return pl.pallas_call(
        _scale1_body,
        grid=(GRID,),
        in_specs=[_xu_spec(0), _part_spec(H, 0), _part_spec(H, 1)],
        out_specs=[_row_spec(H), _row_spec(H)],
        out_shape=[jax.ShapeDtypeStruct((NP, H), jnp.float32),
                   jax.ShapeDtypeStruct((NP, H), jnp.float32)],
    )(xu, deg, deg)


def _layer(xu, part, wh, b_prev, p, y_prev, dinv):
    return pl.pallas_call(
        _layer_body,
        grid=(GRID,),
        in_specs=[_xu_spec(part), _full_spec((H, H)),
                  _full_spec((1, H)), _part_spec(H, 0), _part_spec(H, 1),
                  _row_spec(H), _row_spec(H)],
        out_specs=_row_spec(H),
        out_shape=jax.ShapeDtypeStruct((NP, H), jnp.float32),
    )(xu, wh, b_prev, p, p, y_prev, dinv)


def _final(xu, b4, wl1h, bl1, wl2h, bl2_p, p, y4, dinv):
    return pl.pallas_call(
        _final_body,
        grid=(GRID,),
        in_specs=[_xu_spec(4), _xu_spec(5), _full_spec((1, H)),
                  _full_spec((H, H)), _full_spec((1, H)),
                  _full_spec((H, H)), _full_spec((1, H)),
                  _part_spec(H, 0), _part_spec(H, 1),
                  _row_spec(H), _row_spec(H)],
        out_specs=_row_spec(C),
        out_shape=jax.ShapeDtypeStruct((NP, C), jnp.float32),
    )(xu, xu, b4, wl1h, bl1, wl2h, bl2_p, p, p, y4, dinv)


def kernel(x, edge_index, W1, b1, W2, b2, W3, b3, W4, b4, Wl1, bl1, Wl2, bl2):
    src = edge_index[:, 0]
    dst = edge_index[:, 1]
    pad_idx = jnp.full((E_TOT - E,), N, jnp.int32)
    src_p = jnp.concatenate([src, pad_idx]).reshape(TOTCH + CH_MAX, K)
    dst_p = jnp.concatenate([dst, pad_idx]).reshape(TOTCH + CH_MAX, K)

    x_p = jnp.pad(x, ((0, NP - N), (0, 0)))
    zeros_h = jnp.zeros((NP, H), jnp.float32)
    ones_h = jnp.ones((K, H), jnp.float32)
    wl2_p = jnp.pad(Wl2, ((0, 0), (0, H - C)))
    bl2_p = jnp.pad(bl2, (0, H - C)).reshape(1, H)
    b1r, b2r, b3r, b4r = (b.reshape(1, H) for b in (b1, b2, b3, b4))
    bl1r = bl1.reshape(1, H)

    wx = jnp.stack([W1, W2[:F_IN], W3[:F_IN], W4[:F_IN],
                    Wl1[:F_IN], wl2_p[:F_IN]])

    deg = _sc_hist(dst_p, ones_h, zeros_h)
    xu = _xmm(x_p, wx)
    y1, dinv = _scale1(xu, deg)
    p1 = _sc_prop(y1, src_p, dst_p, zeros_h)
    y2 = _layer(xu, 1, W2[F_IN:], b1r, p1, y1, dinv)
    p2 = _sc_prop(y2, src_p, dst_p, zeros_h)
    y3 = _layer(xu, 2, W3[F_IN:], b2r, p2, y2, dinv)
    p3 = _sc_prop(y3, src_p, dst_p, zeros_h)
    y4 = _layer(xu, 3, W4[F_IN:], b3r, p3, y3, dinv)
    p4 = _sc_prop(y4, src_p, dst_p, zeros_h)
    out = _final(xu, b4r, Wl1[F_IN:], bl1r, wl2_p[F_IN:], bl2_p, p4, y4, dinv)
    return out[:N]


# speedup vs baseline: 1.0772x; 1.0772x over previous
import functools

import jax
import jax.numpy as jnp
from jax import lax
from jax.experimental import pallas as pl
from jax.experimental.pallas import tpu as pltpu
from jax.experimental.pallas import tpu_sc as plsc

N = 10000
E = 160000
F_IN = 256
H = 128
C = 10

NC, NS = 2, 16
NW = NC * NS
K = 128
EPT = ((E + NW * K - 1) // (NW * K)) * K
E_PAD = EPT * NW
NCHUNK = EPT // K
NP = 10112
RPS = NP // NS

BROWS = 1264
GRID = NP // BROWS

_mesh = plsc.VectorSubcoreMesh(core_axis_name="c", subcore_axis_name="s")


NBUF = 2
TOTCH = E_PAD // K
CH_A = 64
CH_B = 16
CH_MAX = max(CH_A, CH_B)
E_TOT = (TOTCH + CH_MAX) * K


@functools.partial(
    pl.kernel,
    out_type=jax.ShapeDtypeStruct((NC, NP, H), jnp.float32),
    mesh=_mesh,
    scratch_types=[
        pltpu.VMEM((NCHUNK, K), jnp.int32),
        pltpu.VMEM((K, H), jnp.float32),
        pltpu.VMEM_SHARED((NP, H), jnp.float32),
    ],
)
def _sc_hist(dst_hbm, ones_hbm, zeros_hbm, out_hbm, dall, ones_v, acc):
    c = lax.axis_index("c")
    s = lax.axis_index("s")
    wid = c * NS + s
    pltpu.sync_copy(zeros_hbm.at[pl.ds(s * RPS, RPS)],
                    acc.at[pl.ds(s * RPS, RPS)])
    pltpu.sync_copy(dst_hbm.at[pl.ds(wid * NCHUNK, NCHUNK)], dall)
    pltpu.sync_copy(ones_hbm, ones_v)
    plsc.subcore_barrier()

    def body(j, carry):
        pltpu.sync_copy(ones_v, acc.at[dall.at[j]], add=True)
        return carry

    lax.fori_loop(0, NCHUNK, body, 0)
    plsc.subcore_barrier()
    pltpu.sync_copy(acc.at[pl.ds(s * RPS, RPS)],
                    out_hbm.at[c, pl.ds(s * RPS, RPS)])


@functools.partial(
    pl.kernel,
    out_type=jax.ShapeDtypeStruct((NC, NP, H), jnp.float32),
    mesh=_mesh,
    scratch_types=[
        pltpu.VMEM((CH_MAX, K), jnp.int32),
        pltpu.VMEM((CH_MAX, K), jnp.int32),
        [pltpu.VMEM((K, H), jnp.float32)] * NBUF,
        pltpu.VMEM_SHARED((NP, H), jnp.float32),
        [pltpu.SemaphoreType.DMA] * NBUF,
    ],
)
def _sc_prop(y_hbm, src_hbm, dst_hbm, zeros_hbm, out_hbm,
             sall, dall, rows, acc, sems):
    c = lax.axis_index("c")
    s = lax.axis_index("s")
    cnt = jnp.where(c == 0, CH_A, CH_B)
    base = jnp.where(c == 0, s * CH_A, NS * CH_A + s * CH_B)
    pltpu.sync_copy(src_hbm.at[pl.ds(base, CH_MAX)], sall)
    pltpu.sync_copy(dst_hbm.at[pl.ds(base, CH_MAX)], dall)
    pltpu.sync_copy(zeros_hbm.at[pl.ds(s * RPS, RPS)],
                    acc.at[pl.ds(s * RPS, RPS)])
    plsc.subcore_barrier()

    for b in range(NBUF):
        pltpu.async_copy(y_hbm.at[sall.at[b]], rows[b], sems[b])

    ngrp = cnt // NBUF

    def grp(g, carry):
        for b in range(NBUF):
            j = g * NBUF + b
            pltpu.make_async_copy(y_hbm.at[sall.at[b]], rows[b],
                                  sems[b]).wait()
            pltpu.sync_copy(rows[b], acc.at[dall.at[j]], add=True)

            @pl.when(g + 1 < ngrp)
            def _():
                pltpu.async_copy(y_hbm.at[sall.at[j + NBUF]], rows[b],
                                 sems[b])
        return carry

    lax.fori_loop(0, ngrp, grp, 0)
    plsc.subcore_barrier()
    pltpu.sync_copy(acc.at[pl.ds(s * RPS, RPS)],
                    out_hbm.at[c, pl.ds(s * RPS, RPS)])


def _mm1_body(x_ref, w_ref, dg0_ref, dg1_ref, y_ref, dinv_ref):
    i = pl.program_id(0)
    d = dg0_ref[0][:, 0:1] + dg1_ref[0][:, 0:1]
    rid = lax.broadcasted_iota(jnp.int32, (BROWS, 1), 0) + i * BROWS
    dv = jnp.where(rid < N, lax.rsqrt(1.0 + d), 0.0)
    y = dv * jnp.dot(x_ref[...], w_ref[...], preferred_element_type=jnp.float32)
    y_ref[...] = y
    dinv_ref[...] = jnp.broadcast_to(dv, (BROWS, H))


def _layer_body(x_ref, w_ref, b_ref, p0_ref, p1_ref, yprev_ref, dinv_ref,
                yout_ref):
    dv = dinv_ref[...]
    h = jnp.maximum(dv * (p0_ref[0] + p1_ref[0] + yprev_ref[...]) + b_ref[...],
                    0.0)
    z = (jnp.dot(x_ref[...], w_ref[0:F_IN, :],
                 preferred_element_type=jnp.float32)
         + jnp.dot(h, w_ref[F_IN:, :], preferred_element_type=jnp.float32))
    yout_ref[...] = dv * z


def _final_body(x_ref, b4_ref, wl1_ref, bl1_ref, wl2_ref, bl2_ref,
                p0_ref, p1_ref, y4_ref, dinv_ref, out_ref):
    dv = dinv_ref[...]
    h4 = jnp.maximum(dv * (p0_ref[0] + p1_ref[0] + y4_ref[...]) + b4_ref[...],
                     0.0)
    h = jnp.maximum(
        jnp.dot(x_ref[...], wl1_ref[0:F_IN, :],
                preferred_element_type=jnp.float32)
        + jnp.dot(h4, wl1_ref[F_IN:, :], preferred_element_type=jnp.float32)
        + bl1_ref[...], 0.0)
    o = (jnp.dot(x_ref[...], wl2_ref[0:F_IN, :],
                 preferred_element_type=jnp.float32)
         + jnp.dot(h, wl2_ref[F_IN:, :], preferred_element_type=jnp.float32)
         + bl2_ref[...])
    col = lax.broadcasted_iota(jnp.int32, (BROWS, H), 1)
    om = jnp.where(col < C, o, -jnp.inf)
    m = jnp.max(om, axis=1, keepdims=True)
    lse = jnp.log(jnp.sum(jnp.exp(om - m), axis=1, keepdims=True))
    out_ref[...] = (om - m - lse)[:, :C]


def _row_spec(cols):
    return pl.BlockSpec((BROWS, cols), lambda i: (i, 0))


def _part_spec(cols, part):
    return pl.BlockSpec((1, BROWS, cols), lambda i, p=part: (p, i, 0))


def _full_spec(shape):
    return pl.BlockSpec(shape, lambda i: tuple(0 for _ in shape))


def _mm1(x_p, w1, deg):
    return pl.pallas_call(
        _mm1_body,
        grid=(GRID,),
        in_specs=[_row_spec(F_IN), _full_spec((F_IN, H)),
                  _part_spec(H, 0), _part_spec(H, 1)],
        out_specs=[_row_spec(H), _row_spec(H)],
        out_shape=[jax.ShapeDtypeStruct((NP, H), jnp.float32),
                   jax.ShapeDtypeStruct((NP, H), jnp.float32)],
    )(x_p, w1, deg, deg)


def _layer(x_p, w, b_prev, p, y_prev, dinv):
    return pl.pallas_call(
        _layer_body,
        grid=(GRID,),
        in_specs=[_row_spec(F_IN), _full_spec((F_IN + H, H)),
                  _full_spec((1, H)), _part_spec(H, 0), _part_spec(H, 1),
                  _row_spec(H), _row_spec(H)],
        out_specs=_row_spec(H),
        out_shape=jax.ShapeDtypeStruct((NP, H), jnp.float32),
    )(x_p, w, b_prev, p, p, y_prev, dinv)


def _final(x_p, b4, wl1, bl1, wl2_p, bl2_p, p, y4, dinv):
    return pl.pallas_call(
        _final_body,
        grid=(GRID,),
        in_specs=[_row_spec(F_IN), _full_spec((1, H)),
                  _full_spec((F_IN + H, H)), _full_spec((1, H)),
                  _full_spec((F_IN + H, H)), _full_spec((1, H)),
                  _part_spec(H, 0), _part_spec(H, 1),
                  _row_spec(H), _row_spec(H)],
        out_specs=_row_spec(C),
        out_shape=jax.ShapeDtypeStruct((NP, C), jnp.float32),
    )(x_p, b4, wl1, bl1, wl2_p, bl2_p, p, p, y4, dinv)


def kernel(x, edge_index, W1, b1, W2, b2, W3, b3, W4, b4, Wl1, bl1, Wl2, bl2):
    src = edge_index[:, 0]
    dst = edge_index[:, 1]
    pad_idx = jnp.full((E_TOT - E,), N, jnp.int32)
    src_p = jnp.concatenate([src, pad_idx]).reshape(TOTCH + CH_MAX, K)
    dst_p = jnp.concatenate([dst, pad_idx]).reshape(TOTCH + CH_MAX, K)

    x_p = jnp.pad(x, ((0, NP - N), (0, 0)))
    zeros_h = jnp.zeros((NP, H), jnp.float32)
    ones_h = jnp.ones((K, H), jnp.float32)
    wl2_p = jnp.pad(Wl2, ((0, 0), (0, H - C)))
    bl2_p = jnp.pad(bl2, (0, H - C)).reshape(1, H)
    b1r, b2r, b3r, b4r = (b.reshape(1, H) for b in (b1, b2, b3, b4))
    bl1r = bl1.reshape(1, H)

    deg = _sc_hist(dst_p, ones_h, zeros_h)
    y1, dinv = _mm1(x_p, W1, deg)
    p1 = _sc_prop(y1, src_p, dst_p, zeros_h)
    y2 = _layer(x_p, W2, b1r, p1, y1, dinv)
    p2 = _sc_prop(y2, src_p, dst_p, zeros_h)
    y3 = _layer(x_p, W3, b2r, p2, y2, dinv)
    p3 = _sc_prop(y3, src_p, dst_p, zeros_h)
    y4 = _layer(x_p, W4, b3r, p3, y3, dinv)
    p4 = _sc_prop(y4, src_p, dst_p, zeros_h)
    out = _final(x_p, b4r, Wl1, bl1r, wl2_p, bl2_p, p4, y4, dinv)
    return out[:N]


# speedup vs baseline: 1.0821x; 1.0045x over previous
import functools

import jax
import jax.numpy as jnp
from jax import lax
from jax.experimental import pallas as pl
from jax.experimental.pallas import tpu as pltpu
from jax.experimental.pallas import tpu_sc as plsc

N = 10000
E = 160000
F_IN = 256
H = 128
C = 10

NC, NS = 2, 16
NW = NC * NS
K = 128
EPT = ((E + NW * K - 1) // (NW * K)) * K
E_PAD = EPT * NW
NCHUNK = EPT // K
NP = 10112
RPS = NP // NS

BROWS = 2528
GRID = NP // BROWS

_mesh = plsc.VectorSubcoreMesh(core_axis_name="c", subcore_axis_name="s")


NBUF = 2
TOTCH = E_PAD // K
CH_A = 64
CH_B = 16
CH_MAX = max(CH_A, CH_B)
E_TOT = (TOTCH + CH_MAX) * K


@functools.partial(
    pl.kernel,
    out_type=jax.ShapeDtypeStruct((NC, NP, H), jnp.float32),
    mesh=_mesh,
    scratch_types=[
        pltpu.VMEM((NCHUNK, K), jnp.int32),
        pltpu.VMEM((K, H), jnp.float32),
        pltpu.VMEM_SHARED((NP, H), jnp.float32),
    ],
)
def _sc_hist(dst_hbm, ones_hbm, zeros_hbm, out_hbm, dall, ones_v, acc):
    c = lax.axis_index("c")
    s = lax.axis_index("s")
    wid = c * NS + s
    pltpu.sync_copy(zeros_hbm.at[pl.ds(s * RPS, RPS)],
                    acc.at[pl.ds(s * RPS, RPS)])
    pltpu.sync_copy(dst_hbm.at[pl.ds(wid * NCHUNK, NCHUNK)], dall)
    pltpu.sync_copy(ones_hbm, ones_v)
    plsc.subcore_barrier()

    def body(j, carry):
        pltpu.sync_copy(ones_v, acc.at[dall.at[j]], add=True)
        return carry

    lax.fori_loop(0, NCHUNK, body, 0)
    plsc.subcore_barrier()
    pltpu.sync_copy(acc.at[pl.ds(s * RPS, RPS)],
                    out_hbm.at[c, pl.ds(s * RPS, RPS)])


@functools.partial(
    pl.kernel,
    out_type=jax.ShapeDtypeStruct((NC, NP, H), jnp.float32),
    mesh=_mesh,
    scratch_types=[
        pltpu.VMEM((CH_MAX, K), jnp.int32),
        pltpu.VMEM((CH_MAX, K), jnp.int32),
        [pltpu.VMEM((K, H), jnp.float32)] * NBUF,
        pltpu.VMEM_SHARED((NP, H), jnp.float32),
        [pltpu.SemaphoreType.DMA] * NBUF,
    ],
)
def _sc_prop(y_hbm, src_hbm, dst_hbm, zeros_hbm, out_hbm,
             sall, dall, rows, acc, sems):
    c = lax.axis_index("c")
    s = lax.axis_index("s")
    cnt = jnp.where(c == 0, CH_A, CH_B)
    base = jnp.where(c == 0, s * CH_A, NS * CH_A + s * CH_B)
    pltpu.sync_copy(src_hbm.at[pl.ds(base, CH_MAX)], sall)
    pltpu.sync_copy(dst_hbm.at[pl.ds(base, CH_MAX)], dall)
    pltpu.sync_copy(zeros_hbm.at[pl.ds(s * RPS, RPS)],
                    acc.at[pl.ds(s * RPS, RPS)])
    plsc.subcore_barrier()

    for b in range(NBUF):
        pltpu.async_copy(y_hbm.at[sall.at[b]], rows[b], sems[b])

    ngrp = cnt // NBUF

    def grp(g, carry):
        for b in range(NBUF):
            j = g * NBUF + b
            pltpu.make_async_copy(y_hbm.at[sall.at[b]], rows[b],
                                  sems[b]).wait()
            pltpu.sync_copy(rows[b], acc.at[dall.at[j]], add=True)

            @pl.when(g + 1 < ngrp)
            def _():
                pltpu.async_copy(y_hbm.at[sall.at[j + NBUF]], rows[b],
                                 sems[b])
        return carry

    lax.fori_loop(0, ngrp, grp, 0)
    plsc.subcore_barrier()
    pltpu.sync_copy(acc.at[pl.ds(s * RPS, RPS)],
                    out_hbm.at[c, pl.ds(s * RPS, RPS)])


def _mm1_body(x_ref, w_ref, dg0_ref, dg1_ref, y_ref, dinv_ref):
    i = pl.program_id(0)
    d = dg0_ref[0][:, 0:1] + dg1_ref[0][:, 0:1]
    rid = lax.broadcasted_iota(jnp.int32, (BROWS, 1), 0) + i * BROWS
    dv = jnp.where(rid < N, lax.rsqrt(1.0 + d), 0.0)
    y = dv * jnp.dot(x_ref[...], w_ref[...], preferred_element_type=jnp.float32)
    y_ref[...] = y
    dinv_ref[...] = jnp.broadcast_to(dv, (BROWS, H))


def _layer_body(x_ref, w_ref, b_ref, p0_ref, p1_ref, yprev_ref, dinv_ref,
                yout_ref):
    dv = dinv_ref[...]
    h = jnp.maximum(dv * (p0_ref[0] + p1_ref[0] + yprev_ref[...]) + b_ref[...],
                    0.0)
    z = (jnp.dot(x_ref[...], w_ref[0:F_IN, :],
                 preferred_element_type=jnp.float32)
         + jnp.dot(h, w_ref[F_IN:, :], preferred_element_type=jnp.float32))
    yout_ref[...] = dv * z


def _final_body(x_ref, b4_ref, wl1_ref, bl1_ref, wl2_ref, bl2_ref,
                p0_ref, p1_ref, y4_ref, dinv_ref, out_ref):
    dv = dinv_ref[...]
    h4 = jnp.maximum(dv * (p0_ref[0] + p1_ref[0] + y4_ref[...]) + b4_ref[...],
                     0.0)
    h = jnp.maximum(
        jnp.dot(x_ref[...], wl1_ref[0:F_IN, :],
                preferred_element_type=jnp.float32)
        + jnp.dot(h4, wl1_ref[F_IN:, :], preferred_element_type=jnp.float32)
        + bl1_ref[...], 0.0)
    o = (jnp.dot(x_ref[...], wl2_ref[0:F_IN, :],
                 preferred_element_type=jnp.float32)
         + jnp.dot(h, wl2_ref[F_IN:, :], preferred_element_type=jnp.float32)
         + bl2_ref[...])
    col = lax.broadcasted_iota(jnp.int32, (BROWS, H), 1)
    om = jnp.where(col < C, o, -jnp.inf)
    m = jnp.max(om, axis=1, keepdims=True)
    lse = jnp.log(jnp.sum(jnp.exp(om - m), axis=1, keepdims=True))
    out_ref[...] = (om - m - lse)[:, :C]


def _row_spec(cols):
    return pl.BlockSpec((BROWS, cols), lambda i: (i, 0))


def _part_spec(cols, part):
    return pl.BlockSpec((1, BROWS, cols), lambda i, p=part: (p, i, 0))


def _full_spec(shape):
    return pl.BlockSpec(shape, lambda i: tuple(0 for _ in shape))


def _mm1(x_p, w1, deg):
    return pl.pallas_call(
        _mm1_body,
        grid=(GRID,),
        in_specs=[_row_spec(F_IN), _full_spec((F_IN, H)),
                  _part_spec(H, 0), _part_spec(H, 1)],
        out_specs=[_row_spec(H), _row_spec(H)],
        out_shape=[jax.ShapeDtypeStruct((NP, H), jnp.float32),
                   jax.ShapeDtypeStruct((NP, H), jnp.float32)],
    )(x_p, w1, deg, deg)


def _layer(x_p, w, b_prev, p, y_prev, dinv):
    return pl.pallas_call(
        _layer_body,
        grid=(GRID,),
        in_specs=[_row_spec(F_IN), _full_spec((F_IN + H, H)),
                  _full_spec((1, H)), _part_spec(H, 0), _part_spec(H, 1),
                  _row_spec(H), _row_spec(H)],
        out_specs=_row_spec(H),
        out_shape=jax.ShapeDtypeStruct((NP, H), jnp.float32),
    )(x_p, w, b_prev, p, p, y_prev, dinv)


def _final(x_p, b4, wl1, bl1, wl2_p, bl2_p, p, y4, dinv):
    return pl.pallas_call(
        _final_body,
        grid=(GRID,),
        in_specs=[_row_spec(F_IN), _full_spec((1, H)),
                  _full_spec((F_IN + H, H)), _full_spec((1, H)),
                  _full_spec((F_IN + H, H)), _full_spec((1, H)),
                  _part_spec(H, 0), _part_spec(H, 1),
                  _row_spec(H), _row_spec(H)],
        out_specs=_row_spec(C),
        out_shape=jax.ShapeDtypeStruct((NP, C), jnp.float32),
    )(x_p, b4, wl1, bl1, wl2_p, bl2_p, p, p, y4, dinv)


def kernel(x, edge_index, W1, b1, W2, b2, W3, b3, W4, b4, Wl1, bl1, Wl2, bl2):
    src = edge_index[:, 0]
    dst = edge_index[:, 1]
    pad_idx = jnp.full((E_TOT - E,), N, jnp.int32)
    src_p = jnp.concatenate([src, pad_idx]).reshape(TOTCH + CH_MAX, K)
    dst_p = jnp.concatenate([dst, pad_idx]).reshape(TOTCH + CH_MAX, K)

    x_p = jnp.pad(x, ((0, NP - N), (0, 0)))
    zeros_h = jnp.zeros((NP, H), jnp.float32)
    ones_h = jnp.ones((K, H), jnp.float32)
    wl2_p = jnp.pad(Wl2, ((0, 0), (0, H - C)))
    bl2_p = jnp.pad(bl2, (0, H - C)).reshape(1, H)
    b1r, b2r, b3r, b4r = (b.reshape(1, H) for b in (b1, b2, b3, b4))
    bl1r = bl1.reshape(1, H)

    deg = _sc_hist(dst_p, ones_h, zeros_h)
    y1, dinv = _mm1(x_p, W1, deg)
    p1 = _sc_prop(y1, src_p, dst_p, zeros_h)
    y2 = _layer(x_p, W2, b1r, p1, y1, dinv)
    p2 = _sc_prop(y2, src_p, dst_p, zeros_h)
    y3 = _layer(x_p, W3, b2r, p2, y2, dinv)
    p3 = _sc_prop(y3, src_p, dst_p, zeros_h)
    y4 = _layer(x_p, W4, b3r, p3, y3, dinv)
    p4 = _sc_prop(y4, src_p, dst_p, zeros_h)
    out = _final(x_p, b4r, Wl1, bl1r, wl2_p, bl2_p, p4, y4, dinv)
    return out[:N]
